# baseline (speedup 1.0000x reference)
import jax
import jax.numpy as jnp
from jax.experimental import pallas as pl

_BR = 256
_CP = 1024


def _fused_body(x_ref, w_ref, mft_ref, out_ref, *, num_classes):
    preds = jnp.dot(x_ref[...], w_ref[...])
    scores = jnp.dot(preds, mft_ref[...])
    mft = mft_ref[...]
    b2 = jnp.sum(mft * mft, axis=0, keepdims=True)
    d2 = b2 - 2.0 * scores
    col = jax.lax.broadcasted_iota(jnp.int32, d2.shape, 1)
    d2 = jnp.where(col < num_classes, d2, jnp.inf)
    rowmin = jnp.min(d2, axis=1, keepdims=True)
    cand = jnp.where(d2 == rowmin, col, d2.shape[1])
    idx = jnp.min(cand, axis=1, keepdims=True)
    ocol = jax.lax.broadcasted_iota(jnp.int32, out_ref.shape, 1)
    out_ref[...] = (ocol == idx).astype(jnp.float32)


def kernel(x, W, mean_features):
    ns, d_in = x.shape
    num_classes, nf = mean_features.shape
    mft = jnp.zeros((nf, _CP), mean_features.dtype).at[:, :num_classes].set(
        mean_features.T)
    import functools
    body = functools.partial(_fused_body, num_classes=num_classes)
    out = pl.pallas_call(
        body,
        grid=(ns // _BR,),
        in_specs=[
            pl.BlockSpec((_BR, d_in), lambda i: (i, 0)),
            pl.BlockSpec((d_in, nf), lambda i: (0, 0)),
            pl.BlockSpec((nf, _CP), lambda i: (0, 0)),
        ],
        out_specs=pl.BlockSpec((_BR, num_classes), lambda i: (i, 0)),
        out_shape=jax.ShapeDtypeStruct((ns, num_classes), jnp.float32),
    )(x, W, mft)
    return out
